# baseline (speedup 1.0000x reference)
import functools

import jax
import jax.numpy as jnp
from jax import lax
from jax.experimental import pallas as pl
from jax.experimental.pallas import tpu as pltpu
from jax.experimental.pallas import tpu_sc as plsc

_N = 10000
_K = 32
_F_IN = 128
_F_OUT = 128

_NC = 2
_NS = 16
_NW = _NC * _NS

_SEGS = (2000, 4000, 4000)
_CH = 80


def _sc_gather_body(table, idx_hbm, out_hbm,
                    xs, idx_v, rows0, rows1, gsem0, gsem1, wsem0, wsem1,
                    seg):
    nchunk = seg // _CH
    sid = lax.axis_index("s")
    wid = sid * _NC + lax.axis_index("c")
    ibase = pl.multiple_of(wid * seg, 8)
    col = pl.multiple_of(wid * _F_IN, _F_IN)
    pltpu.sync_copy(idx_hbm.at[pl.ds(ibase, seg)], idx_v)

    @pl.when(sid == 0)
    def _stage():
        pltpu.sync_copy(table, xs)
    plsc.subcore_barrier()

    bufs = (rows0, rows1)
    gsems = (gsem0, gsem1)
    wsems = (wsem0, wsem1)
    wb = [None, None]
    g = [None, None]

    g[0] = pltpu.async_copy(xs.at[idx_v.at[pl.ds(0, _CH)]], rows0, gsem0)
    for c in range(nchunk):
        b = c & 1
        nb = b ^ 1
        if c + 1 < nchunk:
            if wb[nb] is not None:
                wb[nb].wait()
            g[nb] = pltpu.async_copy(
                xs.at[idx_v.at[pl.ds((c + 1) * _CH, _CH)]],
                bufs[nb], gsems[nb])
        g[b].wait()
        wb[b] = pltpu.async_copy(
            bufs[b],
            out_hbm.at[pl.ds(c * _CH, _CH), pl.ds(col, _F_IN)],
            wsems[b])
    wb[(nchunk - 1) & 1].wait()
    if nchunk > 1:
        wb[nchunk & 1].wait()


@functools.cache
def _sc_gather(seg):
    return pl.kernel(
        functools.partial(_sc_gather_body, seg=seg),
        out_type=jax.ShapeDtypeStruct((seg, _K * _F_IN), jnp.float32),
        mesh=plsc.VectorSubcoreMesh(core_axis_name="c", subcore_axis_name="s"),
        scratch_types=[
            pltpu.VMEM_SHARED((_N, _F_IN), jnp.float32),
            pltpu.VMEM((seg,), jnp.int32),
            pltpu.VMEM((_CH, _F_IN), jnp.float32),
            pltpu.VMEM((_CH, _F_IN), jnp.float32),
            pltpu.SemaphoreType.DMA,
            pltpu.SemaphoreType.DMA,
            pltpu.SemaphoreType.DMA,
            pltpu.SemaphoreType.DMA,
        ],
    )


def _mm_body(xnh_ref, w_ref, b_ref, acc_ref, o_ref):
    del acc_ref
    o_ref[...] = (
        jnp.dot(xnh_ref[...].astype(jnp.bfloat16), w_ref[...],
                preferred_element_type=jnp.float32)
        + b_ref[...]
    )


_ROWS_BLK = 400


def _mm_seg(xnh, w, b, acc, blk_off):
    return pl.pallas_call(
        _mm_body,
        grid=(xnh.shape[0] // _ROWS_BLK,),
        in_specs=[
            pl.BlockSpec((_ROWS_BLK, _K * _F_IN), lambda i: (i, 0)),
            pl.BlockSpec((_K * _F_IN, _F_OUT), lambda i: (0, 0)),
            pl.BlockSpec((1, _F_OUT), lambda i: (0, 0)),
            pl.BlockSpec(memory_space=pl.ANY),
        ],
        out_specs=pl.BlockSpec((_ROWS_BLK, _F_OUT),
                               lambda i, o=blk_off: (i + o, 0)),
        out_shape=jax.ShapeDtypeStruct((_N, _F_OUT), jnp.float32),
        input_output_aliases={3: 0},
    )(xnh, w, b, acc)


def kernel(x, adjc, W, b):
    adjT = adjc.T
    wb = W.astype(jnp.bfloat16)
    b2 = b.reshape(1, _F_OUT)
    acc = jnp.zeros((_N, _F_OUT), jnp.float32)
    off = 0
    for seg in _SEGS:
        idx_s = adjT[:, off:off + seg].reshape(_K * seg)
        xnh = _sc_gather(seg)(x, idx_s)
        acc = _mm_seg(xnh, wb, b2, acc, off // _ROWS_BLK)
        off += seg
    return acc


# speedup vs baseline: 1.0418x; 1.0418x over previous
import functools

import jax
import jax.numpy as jnp
from jax import lax
from jax.experimental import pallas as pl
from jax.experimental.pallas import tpu as pltpu
from jax.experimental.pallas import tpu_sc as plsc

_N = 10000
_K = 32
_F_IN = 128
_F_OUT = 128

_NC = 2
_NS = 16
_NW = _NC * _NS

_SEGS = (10000,)
_CH = 80


def _sc_gather_body(table, idx_hbm, out_hbm,
                    xs, idx_v, rows0, rows1, gsem0, gsem1, wsem0, wsem1,
                    seg):
    nchunk = seg // _CH
    sid = lax.axis_index("s")
    wid = sid * _NC + lax.axis_index("c")
    ibase = pl.multiple_of(wid * seg, 8)
    col = pl.multiple_of(wid * _F_IN, _F_IN)
    pltpu.sync_copy(idx_hbm.at[pl.ds(ibase, seg)], idx_v)

    @pl.when(sid == 0)
    def _stage():
        pltpu.sync_copy(table, xs)
    plsc.subcore_barrier()

    bufs = (rows0, rows1)
    gsems = (gsem0, gsem1)
    wsems = (wsem0, wsem1)
    wb = [None, None]
    g = [None, None]

    g[0] = pltpu.async_copy(xs.at[idx_v.at[pl.ds(0, _CH)]], rows0, gsem0)
    for c in range(nchunk):
        b = c & 1
        nb = b ^ 1
        if c + 1 < nchunk:
            if wb[nb] is not None:
                wb[nb].wait()
            g[nb] = pltpu.async_copy(
                xs.at[idx_v.at[pl.ds((c + 1) * _CH, _CH)]],
                bufs[nb], gsems[nb])
        g[b].wait()
        wb[b] = pltpu.async_copy(
            bufs[b],
            out_hbm.at[pl.ds(c * _CH, _CH), pl.ds(col, _F_IN)],
            wsems[b])
    wb[(nchunk - 1) & 1].wait()
    if nchunk > 1:
        wb[nchunk & 1].wait()


@functools.cache
def _sc_gather(seg):
    return pl.kernel(
        functools.partial(_sc_gather_body, seg=seg),
        out_type=jax.ShapeDtypeStruct((seg, _K * _F_IN), jnp.float32),
        mesh=plsc.VectorSubcoreMesh(core_axis_name="c", subcore_axis_name="s"),
        scratch_types=[
            pltpu.VMEM_SHARED((_N, _F_IN), jnp.float32),
            pltpu.VMEM((seg,), jnp.int32),
            pltpu.VMEM((_CH, _F_IN), jnp.float32),
            pltpu.VMEM((_CH, _F_IN), jnp.float32),
            pltpu.SemaphoreType.DMA,
            pltpu.SemaphoreType.DMA,
            pltpu.SemaphoreType.DMA,
            pltpu.SemaphoreType.DMA,
        ],
    )


def _mm_body(xnh_ref, w_ref, b_ref, acc_ref, o_ref):
    del acc_ref
    o_ref[...] = (
        jnp.dot(xnh_ref[...].astype(jnp.bfloat16), w_ref[...],
                preferred_element_type=jnp.float32)
        + b_ref[...]
    )


_ROWS_BLK = 400


def _mm_seg(xnh, w, b, acc, blk_off):
    return pl.pallas_call(
        _mm_body,
        grid=(xnh.shape[0] // _ROWS_BLK,),
        in_specs=[
            pl.BlockSpec((_ROWS_BLK, _K * _F_IN), lambda i: (i, 0)),
            pl.BlockSpec((_K * _F_IN, _F_OUT), lambda i: (0, 0)),
            pl.BlockSpec((1, _F_OUT), lambda i: (0, 0)),
            pl.BlockSpec(memory_space=pl.ANY),
        ],
        out_specs=pl.BlockSpec((_ROWS_BLK, _F_OUT),
                               lambda i, o=blk_off: (i + o, 0)),
        out_shape=jax.ShapeDtypeStruct((_N, _F_OUT), jnp.float32),
        input_output_aliases={3: 0},
    )(xnh, w, b, acc)


def kernel(x, adjc, W, b):
    adjT = adjc.T
    wb = W.astype(jnp.bfloat16)
    b2 = b.reshape(1, _F_OUT)
    acc = jnp.zeros((_N, _F_OUT), jnp.float32)
    off = 0
    for seg in _SEGS:
        idx_s = adjT[:, off:off + seg].reshape(_K * seg)
        xnh = _sc_gather(seg)(x, idx_s)
        acc = _mm_seg(xnh, wb, b2, acc, off // _ROWS_BLK)
        off += seg
    return acc
